# baseline (speedup 1.0000x reference)
import functools

import jax
import jax.numpy as jnp
from jax import lax
from jax.experimental import pallas as pl
from jax.experimental.pallas import tpu as pltpu
from jax.experimental.pallas import tpu_sc as plsc

NC = 2
NS = 16


def _make_deg_kernel(E, NP, K, C):
    sp = NP // NS
    nrows = E // K
    nouter = nrows // (NC * NS * C)
    mesh = plsc.VectorSubcoreMesh(core_axis_name="c", subcore_axis_name="s")

    @functools.partial(
        pl.kernel,
        out_type=jax.ShapeDtypeStruct((NC * NP,), jnp.float32),
        mesh=mesh,
        scratch_types=[
            pltpu.VMEM((C, K), jnp.int32),
            pltpu.VMEM((K,), jnp.float32),
            pltpu.VMEM((sp,), jnp.float32),
            pltpu.VMEM_SHARED((NP,), jnp.float32),
            pltpu.SemaphoreType.DMA,
        ],
        compiler_params=pltpu.CompilerParams(use_tc_tiling_on_sc=False),
    )
    def deg_k(ei3_hbm, ones_hbm, zeros_hbm, deg_hbm,
              idx_v, ones_v, zv, acc_sh, sem):
        cid = lax.axis_index("c")
        sid = lax.axis_index("s")
        pltpu.sync_copy(ones_hbm, ones_v)
        pltpu.sync_copy(zeros_hbm, zv)
        pltpu.sync_copy(zv, acc_sh.at[pl.ds(sid * sp, sp)])
        plsc.subcore_barrier()

        def chunk(o, _):
            base = pl.multiple_of(
                cid * (nrows // NC) + (sid * nouter + o) * C, C)
            pltpu.sync_copy(ei3_hbm.at[1, pl.ds(base, C), :], idx_v)
            hs = [
                pltpu.async_copy(ones_v, acc_sh.at[idx_v.at[j]], sem,
                                 add=True)
                for j in range(C)
            ]
            for h in hs:
                h.wait()
            return 0

        lax.fori_loop(0, nouter, chunk, 0)
        plsc.subcore_barrier()
        pltpu.sync_copy(acc_sh.at[pl.ds(sid * sp, sp)], zv)
        pltpu.sync_copy(zv, deg_hbm.at[pl.ds(cid * NP + sid * sp, sp)])

    return deg_k


def _make_agg_kernel(E, N, Dh, K, C, ZR):
    sp = N // NS
    nrows = E // K
    nouter = nrows // (NS * C)
    mesh = plsc.VectorSubcoreMesh(core_axis_name="c", subcore_axis_name="s")

    @functools.partial(
        pl.kernel,
        out_type=jax.ShapeDtypeStruct((NC * N, Dh), jnp.float32),
        mesh=mesh,
        scratch_types=(
            [pltpu.VMEM((C, K), jnp.int32)] * 2
            + [pltpu.VMEM((K, Dh), jnp.float32)] * C
            + [pltpu.VMEM((ZR, Dh), jnp.float32),
               pltpu.VMEM_SHARED((N, Dh), jnp.float32)]
            + [pltpu.SemaphoreType.DMA] * C
        ),
        compiler_params=pltpu.CompilerParams(use_tc_tiling_on_sc=False),
    )
    def agg_k(ei3_hbm, yC_hbm, zeros_hbm, agg_hbm, *scr):
        ridx_v, cidx_v = scr[0], scr[1]
        bufs = scr[2:2 + C]
        zeros_v, acc_sh = scr[2 + C], scr[3 + C]
        gsems = scr[4 + C:4 + 2 * C]
        cid = lax.axis_index("c")
        sid = lax.axis_index("s")
        pltpu.sync_copy(zeros_hbm, zeros_v)
        for q in range(sp // ZR):
            pltpu.sync_copy(zeros_v, acc_sh.at[pl.ds(sid * sp + q * ZR, ZR), :])
        plsc.subcore_barrier()

        yoff = cid * N

        def chunk(o, _):
            base = pl.multiple_of((sid * nouter + o) * C, C)
            pltpu.sync_copy(ei3_hbm.at[0, pl.ds(base, C), :], ridx_v)
            pltpu.sync_copy(ei3_hbm.at[1, pl.ds(base, C), :], cidx_v)
            nfull = K // 16
            tail_new = K - nfull * 16
            for j in range(C):
                for i in range(nfull):
                    v = ridx_v[j, pl.ds(i * 16, 16)]
                    ridx_v[j, pl.ds(i * 16, 16)] = v + yoff
                if tail_new:
                    v = ridx_v[j, pl.ds(K - 16, 16)]
                    m = lax.iota(jnp.int32, 16) >= (16 - tail_new)
                    ridx_v[j, pl.ds(K - 16, 16)] = v + jnp.where(m, yoff, 0)
            gs = [
                pltpu.async_copy(yC_hbm.at[ridx_v.at[j]], bufs[j], gsems[j])
                for j in range(C)
            ]
            for j in range(C):
                gs[j].wait()
                pltpu.sync_copy(bufs[j], acc_sh.at[cidx_v.at[j]], add=True)
            return 0

        lax.fori_loop(0, nouter, chunk, 0)
        plsc.subcore_barrier()
        for q in range(sp // ZR):
            pltpu.sync_copy(acc_sh.at[pl.ds(sid * sp + q * ZR, ZR), :], zeros_v)
            pltpu.sync_copy(
                zeros_v,
                agg_hbm.at[pl.ds(cid * N + sid * sp + q * ZR, ZR), :],
            )

    return agg_k


def _y_body(d0_ref, d1_ref, x_ref, y_ref, dis_ref):
    c = pl.program_id(0)
    dis = lax.rsqrt(d0_ref[...] + d1_ref[...] + 1.0)
    xb = x_ref[...]
    Dh = xb.shape[1] // 2
    xh = jnp.where(c == 0, xb[:, :Dh], xb[:, Dh:])
    y_ref[...] = xh * dis
    dis_ref[...] = dis


def _out_body(dis_ref, x_ref, a0_ref, a1_ref, W_ref, b_ref, g_ref, be_ref,
              o_ref):
    dis = dis_ref[...]
    agg = jnp.concatenate([a0_ref[...], a1_ref[...]], axis=1)
    z = (agg + x_ref[...] * dis) * dis
    h = lax.dot_general(z, W_ref[...], (((1,), (1,)), ((), ())),
                        preferred_element_type=jnp.float32)
    h = h + b_ref[...][None, :]
    mu = jnp.mean(h, axis=1, keepdims=True)
    var = jnp.mean((h - mu) ** 2, axis=1, keepdims=True)
    o_ref[...] = (h - mu) * lax.rsqrt(var + 1e-5) * g_ref[...][None, :] \
        + be_ref[...][None, :]


def kernel(x, edge_index, W, b, gamma, beta):
    N, D = x.shape
    E = edge_index.shape[1]
    Dh = D // 2
    K = 50
    C = 10
    NP = ((N + 8 * NS - 1) // (8 * NS)) * (8 * NS)
    assert N % NS == 0 and E % (NS * K * C) == 0 and E % (NC * NS * K * C) == 0
    sp = N // NS
    ZR = 125
    assert sp % ZR == 0

    ei3 = edge_index.astype(jnp.int32).reshape(2, E // K, K)
    onesK = jnp.ones((K,), jnp.float32)
    zeros1 = jnp.zeros((NP // NS,), jnp.float32)
    zeros2 = jnp.zeros((ZR, Dh), jnp.float32)

    degp = _make_deg_kernel(E, NP, K, C)(ei3, onesK, zeros1)
    d0 = lax.slice(degp, (0,), (N,)).reshape(N, 1)
    d1 = lax.slice(degp, (NP,), (NP + N,)).reshape(N, 1)

    RB = 2000
    nb = N // RB
    assert N % RB == 0
    yC, dis = pl.pallas_call(
        _y_body,
        grid=(NC, nb),
        in_specs=[
            pl.BlockSpec((RB, 1), lambda c, i: (i, 0)),
            pl.BlockSpec((RB, 1), lambda c, i: (i, 0)),
            pl.BlockSpec((RB, D), lambda c, i: (i, 0)),
        ],
        out_specs=[
            pl.BlockSpec((RB, Dh), lambda c, i: (c * nb + i, 0)),
            pl.BlockSpec((RB, 1), lambda c, i: (i, 0)),
        ],
        out_shape=[
            jax.ShapeDtypeStruct((NC * N, Dh), jnp.float32),
            jax.ShapeDtypeStruct((N, 1), jnp.float32),
        ],
    )(d0, d1, x)

    aggp = _make_agg_kernel(E, N, Dh, K, C, ZR)(ei3, yC, zeros2)

    out = pl.pallas_call(
        _out_body,
        grid=(nb,),
        in_specs=[
            pl.BlockSpec((RB, 1), lambda i: (i, 0)),
            pl.BlockSpec((RB, D), lambda i: (i, 0)),
            pl.BlockSpec((RB, Dh), lambda i: (i, 0)),
            pl.BlockSpec((RB, Dh), lambda i: (nb + i, 0)),
            pl.BlockSpec((D, D), lambda i: (0, 0)),
            pl.BlockSpec((D,), lambda i: (0,)),
            pl.BlockSpec((D,), lambda i: (0,)),
            pl.BlockSpec((D,), lambda i: (0,)),
        ],
        out_specs=pl.BlockSpec((RB, D), lambda i: (i, 0)),
        out_shape=jax.ShapeDtypeStruct((N, D), jnp.float32),
    )(dis, x, aggp, aggp, W, b, gamma, beta)
    return out


# speedup vs baseline: 1.0719x; 1.0719x over previous
import functools

import jax
import jax.numpy as jnp
from jax import lax
from jax.experimental import pallas as pl
from jax.experimental.pallas import tpu as pltpu
from jax.experimental.pallas import tpu_sc as plsc

NC = 2
NS = 16


def _make_deg_kernel(E, NP, K, C):
    sp = NP // NS
    nrows = E // K
    nouter = nrows // (NC * NS * C)
    mesh = plsc.VectorSubcoreMesh(core_axis_name="c", subcore_axis_name="s")

    @functools.partial(
        pl.kernel,
        out_type=jax.ShapeDtypeStruct((NC * NP,), jnp.float32),
        mesh=mesh,
        scratch_types=[
            pltpu.VMEM((C, K), jnp.int32),
            pltpu.VMEM((K,), jnp.float32),
            pltpu.VMEM((sp,), jnp.float32),
            pltpu.VMEM_SHARED((NP,), jnp.float32),
            pltpu.SemaphoreType.DMA,
        ],
        compiler_params=pltpu.CompilerParams(use_tc_tiling_on_sc=False),
    )
    def deg_k(ei3_hbm, ones_hbm, zeros_hbm, deg_hbm,
              idx_v, ones_v, zv, acc_sh, sem):
        cid = lax.axis_index("c")
        sid = lax.axis_index("s")
        pltpu.sync_copy(ones_hbm, ones_v)
        pltpu.sync_copy(zeros_hbm, zv)
        pltpu.sync_copy(zv, acc_sh.at[pl.ds(sid * sp, sp)])
        plsc.subcore_barrier()

        def chunk(o, _):
            base = pl.multiple_of(
                cid * (nrows // NC) + (sid * nouter + o) * C, C)
            pltpu.sync_copy(ei3_hbm.at[1, pl.ds(base, C), :], idx_v)
            hs = [
                pltpu.async_copy(ones_v, acc_sh.at[idx_v.at[j]], sem,
                                 add=True)
                for j in range(C)
            ]
            for h in hs:
                h.wait()
            return 0

        lax.fori_loop(0, nouter, chunk, 0)
        plsc.subcore_barrier()
        pltpu.sync_copy(acc_sh.at[pl.ds(sid * sp, sp)], zv)
        pltpu.sync_copy(zv, deg_hbm.at[pl.ds(cid * NP + sid * sp, sp)])

    return deg_k


def _make_agg_kernel(E, N, Dh, K, C, ZR):
    sp = N // NS
    nrows = E // K
    nouter = nrows // (NS * C)
    mesh = plsc.VectorSubcoreMesh(core_axis_name="c", subcore_axis_name="s")

    @functools.partial(
        pl.kernel,
        out_type=jax.ShapeDtypeStruct((NC * N, Dh), jnp.float32),
        mesh=mesh,
        scratch_types=(
            [pltpu.VMEM((C, K), jnp.int32)] * 2
            + [pltpu.VMEM((K, Dh), jnp.float32)] * C
            + [pltpu.VMEM((ZR, Dh), jnp.float32),
               pltpu.VMEM_SHARED((N, Dh), jnp.float32)]
            + [pltpu.SemaphoreType.DMA] * C
        ),
        compiler_params=pltpu.CompilerParams(use_tc_tiling_on_sc=False),
    )
    def agg_k(ei3_hbm, yC_hbm, zeros_hbm, agg_hbm, *scr):
        ridx_v, cidx_v = scr[0], scr[1]
        bufs = scr[2:2 + C]
        zeros_v, acc_sh = scr[2 + C], scr[3 + C]
        gsems = scr[4 + C:4 + 2 * C]
        cid = lax.axis_index("c")
        sid = lax.axis_index("s")
        pltpu.sync_copy(zeros_hbm, zeros_v)
        for q in range(sp // ZR):
            pltpu.sync_copy(zeros_v, acc_sh.at[pl.ds(sid * sp + q * ZR, ZR), :])
        plsc.subcore_barrier()

        yoff = cid * N

        def chunk(o, _):
            base = pl.multiple_of((sid * nouter + o) * C, C)
            pltpu.sync_copy(ei3_hbm.at[0, pl.ds(base, C), :], ridx_v)
            pltpu.sync_copy(ei3_hbm.at[1, pl.ds(base, C), :], cidx_v)
            nfull = K // 16
            tail_new = K - nfull * 16
            for j in range(C):
                for i in range(nfull):
                    v = ridx_v[j, pl.ds(i * 16, 16)]
                    ridx_v[j, pl.ds(i * 16, 16)] = v + yoff
                if tail_new:
                    v = ridx_v[j, pl.ds(K - 16, 16)]
                    m = lax.iota(jnp.int32, 16) >= (16 - tail_new)
                    ridx_v[j, pl.ds(K - 16, 16)] = v + jnp.where(m, yoff, 0)
            gs = [
                pltpu.async_copy(yC_hbm.at[ridx_v.at[j]], bufs[j], gsems[j])
                for j in range(C)
            ]
            for j in range(C):
                gs[j].wait()
                pltpu.sync_copy(bufs[j], acc_sh.at[cidx_v.at[j]], add=True)
            return 0

        lax.fori_loop(0, nouter, chunk, 0)
        plsc.subcore_barrier()
        for q in range(sp // ZR):
            pltpu.sync_copy(acc_sh.at[pl.ds(sid * sp + q * ZR, ZR), :], zeros_v)
            pltpu.sync_copy(
                zeros_v,
                agg_hbm.at[pl.ds(cid * N + sid * sp + q * ZR, ZR), :],
            )

    return agg_k


def _y_body(d0_ref, d1_ref, x_ref, y_ref, dis_ref):
    c = pl.program_id(0)
    dis = lax.rsqrt(d0_ref[...] + d1_ref[...] + 1.0)
    xb = x_ref[...]
    Dh = xb.shape[1] // 2
    xh = jnp.where(c == 0, xb[:, :Dh], xb[:, Dh:])
    y_ref[...] = xh * dis
    dis_ref[...] = dis


def _out_body(dis_ref, x_ref, a0_ref, a1_ref, W_ref, b_ref, g_ref, be_ref,
              o_ref):
    dis = dis_ref[...]
    agg = jnp.concatenate([a0_ref[...], a1_ref[...]], axis=1)
    z = (agg + x_ref[...] * dis) * dis
    h = lax.dot_general(z, W_ref[...], (((1,), (1,)), ((), ())),
                        preferred_element_type=jnp.float32)
    h = h + b_ref[...][None, :]
    mu = jnp.mean(h, axis=1, keepdims=True)
    var = jnp.mean((h - mu) ** 2, axis=1, keepdims=True)
    o_ref[...] = (h - mu) * lax.rsqrt(var + 1e-5) * g_ref[...][None, :] \
        + be_ref[...][None, :]


def kernel(x, edge_index, W, b, gamma, beta):
    N, D = x.shape
    E = edge_index.shape[1]
    Dh = D // 2
    K = 125
    C = 4
    NP = ((N + 8 * NS - 1) // (8 * NS)) * (8 * NS)
    assert N % NS == 0 and E % (NS * K * C) == 0 and E % (NC * NS * K * C) == 0
    sp = N // NS
    ZR = 125
    assert sp % ZR == 0

    ei3 = edge_index.astype(jnp.int32).reshape(2, E // K, K)
    onesK = jnp.ones((K,), jnp.float32)
    zeros1 = jnp.zeros((NP // NS,), jnp.float32)
    zeros2 = jnp.zeros((ZR, Dh), jnp.float32)

    degp = _make_deg_kernel(E, NP, K, C)(ei3, onesK, zeros1)
    d0 = lax.slice(degp, (0,), (N,)).reshape(N, 1)
    d1 = lax.slice(degp, (NP,), (NP + N,)).reshape(N, 1)

    RB = 2000
    nb = N // RB
    assert N % RB == 0
    yC, dis = pl.pallas_call(
        _y_body,
        grid=(NC, nb),
        in_specs=[
            pl.BlockSpec((RB, 1), lambda c, i: (i, 0)),
            pl.BlockSpec((RB, 1), lambda c, i: (i, 0)),
            pl.BlockSpec((RB, D), lambda c, i: (i, 0)),
        ],
        out_specs=[
            pl.BlockSpec((RB, Dh), lambda c, i: (c * nb + i, 0)),
            pl.BlockSpec((RB, 1), lambda c, i: (i, 0)),
        ],
        out_shape=[
            jax.ShapeDtypeStruct((NC * N, Dh), jnp.float32),
            jax.ShapeDtypeStruct((N, 1), jnp.float32),
        ],
    )(d0, d1, x)

    aggp = _make_agg_kernel(E, N, Dh, K, C, ZR)(ei3, yC, zeros2)

    out = pl.pallas_call(
        _out_body,
        grid=(nb,),
        in_specs=[
            pl.BlockSpec((RB, 1), lambda i: (i, 0)),
            pl.BlockSpec((RB, D), lambda i: (i, 0)),
            pl.BlockSpec((RB, Dh), lambda i: (i, 0)),
            pl.BlockSpec((RB, Dh), lambda i: (nb + i, 0)),
            pl.BlockSpec((D, D), lambda i: (0, 0)),
            pl.BlockSpec((D,), lambda i: (0,)),
            pl.BlockSpec((D,), lambda i: (0,)),
            pl.BlockSpec((D,), lambda i: (0,)),
        ],
        out_specs=pl.BlockSpec((RB, D), lambda i: (i, 0)),
        out_shape=jax.ShapeDtypeStruct((N, D), jnp.float32),
    )(dis, x, aggp, aggp, W, b, gamma, beta)
    return out


# speedup vs baseline: 1.0827x; 1.0101x over previous
import functools

import jax
import jax.numpy as jnp
from jax import lax
from jax.experimental import pallas as pl
from jax.experimental.pallas import tpu as pltpu
from jax.experimental.pallas import tpu_sc as plsc

NC = 2
NS = 16


def _make_deg_kernel(E, NP, K, C):
    sp = NP // NS
    nrows = E // K
    nouter = nrows // (NC * NS * C)
    mesh = plsc.VectorSubcoreMesh(core_axis_name="c", subcore_axis_name="s")

    @functools.partial(
        pl.kernel,
        out_type=jax.ShapeDtypeStruct((NC * NP,), jnp.float32),
        mesh=mesh,
        scratch_types=[
            pltpu.VMEM((C, K), jnp.int32),
            pltpu.VMEM((K,), jnp.float32),
            pltpu.VMEM((sp,), jnp.float32),
            pltpu.VMEM_SHARED((NP,), jnp.float32),
            pltpu.SemaphoreType.DMA,
        ],
        compiler_params=pltpu.CompilerParams(use_tc_tiling_on_sc=False),
    )
    def deg_k(ei3_hbm, ones_hbm, zeros_hbm, deg_hbm,
              idx_v, ones_v, zv, acc_sh, sem):
        cid = lax.axis_index("c")
        sid = lax.axis_index("s")
        pltpu.sync_copy(ones_hbm, ones_v)
        pltpu.sync_copy(zeros_hbm, zv)
        pltpu.sync_copy(zv, acc_sh.at[pl.ds(sid * sp, sp)])
        plsc.subcore_barrier()

        def chunk(o, _):
            base = pl.multiple_of(
                cid * (nrows // NC) + (sid * nouter + o) * C, C)
            pltpu.sync_copy(ei3_hbm.at[1, pl.ds(base, C), :], idx_v)
            hs = [
                pltpu.async_copy(ones_v, acc_sh.at[idx_v.at[j]], sem,
                                 add=True)
                for j in range(C)
            ]
            for h in hs:
                h.wait()
            return 0

        lax.fori_loop(0, nouter, chunk, 0)
        plsc.subcore_barrier()
        pltpu.sync_copy(acc_sh.at[pl.ds(sid * sp, sp)], zv)
        pltpu.sync_copy(zv, deg_hbm.at[pl.ds(cid * NP + sid * sp, sp)])

    return deg_k


def _make_agg_kernel(E, N, Dh, K, C, ZR):
    sp = N // NS
    nrows = E // K
    nouter = nrows // (NS * C)
    mesh = plsc.VectorSubcoreMesh(core_axis_name="c", subcore_axis_name="s")

    @functools.partial(
        pl.kernel,
        out_type=jax.ShapeDtypeStruct((NC * N, Dh), jnp.float32),
        mesh=mesh,
        scratch_types=(
            [pltpu.VMEM((C, K), jnp.int32)] * 2
            + [pltpu.VMEM((K, Dh), jnp.float32)] * C
            + [pltpu.VMEM((ZR, Dh), jnp.float32),
               pltpu.VMEM_SHARED((N, Dh), jnp.float32)]
            + [pltpu.SemaphoreType.DMA] * C
        ),
        compiler_params=pltpu.CompilerParams(use_tc_tiling_on_sc=False),
    )
    def agg_k(ei3_hbm, yC_hbm, zeros_hbm, agg_hbm, *scr):
        ridx_v, cidx_v = scr[0], scr[1]
        bufs = scr[2:2 + C]
        zeros_v, acc_sh = scr[2 + C], scr[3 + C]
        gsems = scr[4 + C:4 + 2 * C]
        cid = lax.axis_index("c")
        sid = lax.axis_index("s")
        pltpu.sync_copy(zeros_hbm, zeros_v)
        for q in range(sp // ZR):
            pltpu.sync_copy(zeros_v, acc_sh.at[pl.ds(sid * sp + q * ZR, ZR), :])
        plsc.subcore_barrier()

        yoff = cid * N

        def chunk(o, _):
            base = pl.multiple_of((sid * nouter + o) * C, C)
            pltpu.sync_copy(ei3_hbm.at[0, pl.ds(base, C), :], ridx_v)
            pltpu.sync_copy(ei3_hbm.at[1, pl.ds(base, C), :], cidx_v)
            nfull = K // 16
            tail_new = K - nfull * 16
            for j in range(C):
                for i in range(nfull):
                    v = ridx_v[j, pl.ds(i * 16, 16)]
                    ridx_v[j, pl.ds(i * 16, 16)] = v + yoff
                if tail_new:
                    v = ridx_v[j, pl.ds(K - 16, 16)]
                    m = lax.iota(jnp.int32, 16) >= (16 - tail_new)
                    ridx_v[j, pl.ds(K - 16, 16)] = v + jnp.where(m, yoff, 0)
            gs = [
                pltpu.async_copy(yC_hbm.at[ridx_v.at[j]], bufs[j], gsems[j])
                for j in range(C)
            ]
            for j in range(C):
                gs[j].wait()
                pltpu.sync_copy(bufs[j], acc_sh.at[cidx_v.at[j]], add=True)
            return 0

        lax.fori_loop(0, nouter, chunk, 0)
        plsc.subcore_barrier()
        for q in range(sp // ZR):
            pltpu.sync_copy(acc_sh.at[pl.ds(sid * sp + q * ZR, ZR), :], zeros_v)
            pltpu.sync_copy(
                zeros_v,
                agg_hbm.at[pl.ds(cid * N + sid * sp + q * ZR, ZR), :],
            )

    return agg_k


def _y_body(d0_ref, d1_ref, x_ref, y_ref):
    c = pl.program_id(0)
    dis = lax.rsqrt(d0_ref[...] + d1_ref[...] + 1.0)
    xb = x_ref[...]
    Dh = xb.shape[1] // 2
    xh = jnp.where(c == 0, xb[:, :Dh], xb[:, Dh:])
    y_ref[...] = xh * dis


def _out_body(d0_ref, d1_ref, x_ref, a0_ref, a1_ref, W_ref, b_ref, g_ref,
              be_ref, o_ref):
    dis = lax.rsqrt(d0_ref[...] + d1_ref[...] + 1.0)
    agg = jnp.concatenate([a0_ref[...], a1_ref[...]], axis=1)
    z = (agg + x_ref[...] * dis) * dis
    h = lax.dot_general(z, W_ref[...], (((1,), (1,)), ((), ())),
                        preferred_element_type=jnp.float32)
    h = h + b_ref[...][None, :]
    mu = jnp.mean(h, axis=1, keepdims=True)
    var = jnp.mean((h - mu) ** 2, axis=1, keepdims=True)
    o_ref[...] = (h - mu) * lax.rsqrt(var + 1e-5) * g_ref[...][None, :] \
        + be_ref[...][None, :]


def kernel(x, edge_index, W, b, gamma, beta):
    N, D = x.shape
    E = edge_index.shape[1]
    Dh = D // 2
    K = 125
    C = 4
    NP = ((N + 8 * NS - 1) // (8 * NS)) * (8 * NS)
    assert N % NS == 0 and E % (NS * K * C) == 0 and E % (NC * NS * K * C) == 0
    sp = N // NS
    ZR = 125
    assert sp % ZR == 0

    ei3 = edge_index.astype(jnp.int32).reshape(2, E // K, K)
    onesK = jnp.ones((K,), jnp.float32)
    zeros1 = jnp.zeros((NP // NS,), jnp.float32)
    zeros2 = jnp.zeros((ZR, Dh), jnp.float32)

    degp = _make_deg_kernel(E, NP, K, C)(ei3, onesK, zeros1)
    d0 = lax.slice(degp, (0,), (N,)).reshape(N, 1)
    d1 = lax.slice(degp, (NP,), (NP + N,)).reshape(N, 1)

    RB = 2000
    nb = N // RB
    assert N % RB == 0
    yC = pl.pallas_call(
        _y_body,
        grid=(NC, nb),
        in_specs=[
            pl.BlockSpec((RB, 1), lambda c, i: (i, 0)),
            pl.BlockSpec((RB, 1), lambda c, i: (i, 0)),
            pl.BlockSpec((RB, D), lambda c, i: (i, 0)),
        ],
        out_specs=pl.BlockSpec((RB, Dh), lambda c, i: (c * nb + i, 0)),
        out_shape=jax.ShapeDtypeStruct((NC * N, Dh), jnp.float32),
    )(d0, d1, x)

    aggp = _make_agg_kernel(E, N, Dh, K, C, ZR)(ei3, yC, zeros2)

    out = pl.pallas_call(
        _out_body,
        grid=(nb,),
        in_specs=[
            pl.BlockSpec((RB, 1), lambda i: (i, 0)),
            pl.BlockSpec((RB, 1), lambda i: (i, 0)),
            pl.BlockSpec((RB, D), lambda i: (i, 0)),
            pl.BlockSpec((RB, Dh), lambda i: (i, 0)),
            pl.BlockSpec((RB, Dh), lambda i: (nb + i, 0)),
            pl.BlockSpec((D, D), lambda i: (0, 0)),
            pl.BlockSpec((D,), lambda i: (0,)),
            pl.BlockSpec((D,), lambda i: (0,)),
            pl.BlockSpec((D,), lambda i: (0,)),
        ],
        out_specs=pl.BlockSpec((RB, D), lambda i: (i, 0)),
        out_shape=jax.ShapeDtypeStruct((N, D), jnp.float32),
    )(d0, d1, x, aggp, aggp, W, b, gamma, beta)
    return out
